# Initial kernel scaffold; baseline (speedup 1.0000x reference)
import jax
import jax.numpy as jnp
from jax.experimental import pallas as pl


def kernel(input_ids, table):
    raise NotImplementedError("write your pallas kernel here")


# speedup vs baseline: 2.1218x; 2.1218x over previous
import math
import numpy as np
import jax
import jax.numpy as jnp
from jax import lax
from jax.experimental import pallas as pl
from jax.experimental.pallas import tpu as pltpu
from jax.experimental.pallas import tpu_sc as plsc

VOCAB = 100000
D = 128
S = 200
B = 4096

NC = 2
NS = 16
NW = NC * NS
ROWS = B * S
RPW = ROWS // NW
CHUNK = 128
NCHUNKS = RPW // CHUNK
LANES = 16
DV = D // LANES


def _make_pos_enc():
    pos_enc = np.zeros((S, D), dtype=np.float32)
    position = np.arange(0, S, dtype=np.float32)[:, None]
    div_term = np.exp(
        np.arange(0, D, 2, dtype=np.float32) * (-math.log(10000.0) / D))
    pos_enc[:, 0::2] = np.sin(position * div_term)
    pos_enc[:, 1::2] = np.cos(position * div_term)
    return jnp.asarray(pos_enc)


_MESH = plsc.VectorSubcoreMesh(core_axis_name="c", subcore_axis_name="s")


def _body(ids_hbm, table_hbm, pe_hbm, out_hbm, idx_v, buf, pe_v, sem):
    wid = lax.axis_index("s") * NC + lax.axis_index("c")
    base = wid * RPW
    pltpu.sync_copy(ids_hbm.at[wid], idx_v)
    pltpu.sync_copy(pe_hbm, pe_v)

    def chunk_step(j, carry):
        pltpu.async_copy(table_hbm.at[idx_v.at[j]], buf, sem).wait()

        def row_step(k, carry2):
            p = lax.rem(j * CHUNK + k, S)
            for c in range(DV):
                sl = pl.ds(c * LANES, LANES)
                buf[k, sl] = buf[k, sl] + pe_v[p, sl]
            return carry2

        lax.fori_loop(0, CHUNK, row_step, 0, unroll=False)
        pltpu.sync_copy(buf, out_hbm.at[pl.ds(base + j * CHUNK, CHUNK)])
        return carry

    lax.fori_loop(0, NCHUNKS, chunk_step, 0, unroll=False)


_sc_call = pl.kernel(
    _body,
    out_type=jax.ShapeDtypeStruct((ROWS, D), jnp.float32),
    mesh=_MESH,
    scratch_types=[
        pltpu.VMEM((NCHUNKS, CHUNK), jnp.int32),
        pltpu.VMEM((CHUNK, D), jnp.float32),
        pltpu.VMEM((S, D), jnp.float32),
        pltpu.SemaphoreType.DMA,
    ],
)


def kernel(input_ids, table):
    ids = input_ids.astype(jnp.int32).reshape(NW, NCHUNKS, CHUNK)
    pe = _make_pos_enc()
    out = _sc_call(ids, table, pe)
    return out.reshape(B, S, D)


# speedup vs baseline: 3.3939x; 1.5996x over previous
import math
import numpy as np
import jax
import jax.numpy as jnp
from jax import lax
from jax.experimental import pallas as pl
from jax.experimental.pallas import tpu as pltpu
from jax.experimental.pallas import tpu_sc as plsc

VOCAB = 100000
D = 128
S = 200
B = 4096

NC = 2
NS = 16
NW = NC * NS
ROWS = B * S
RPW = ROWS // NW
CHUNK = 128
NCHUNKS = RPW // CHUNK
LANES = 16
DV = D // LANES


def _make_pos_enc():
    pos_enc = np.zeros((S, D), dtype=np.float32)
    position = np.arange(0, S, dtype=np.float32)[:, None]
    div_term = np.exp(
        np.arange(0, D, 2, dtype=np.float32) * (-math.log(10000.0) / D))
    pos_enc[:, 0::2] = np.sin(position * div_term)
    pos_enc[:, 1::2] = np.cos(position * div_term)
    return jnp.asarray(pos_enc)


_MESH = plsc.VectorSubcoreMesh(core_axis_name="c", subcore_axis_name="s")


NPAIR = NCHUNKS // 2


def _body(ids_hbm, table_hbm, pe_hbm, out_hbm, idx_v, pe_v, buf0, buf1,
          gs0, gs1, ws0, ws1):
    wid = lax.axis_index("s") * NC + lax.axis_index("c")
    base = wid * RPW
    pltpu.sync_copy(ids_hbm.at[wid], idx_v)
    pltpu.sync_copy(pe_hbm, pe_v)

    def gather_start(j, buf, sem):
        pltpu.async_copy(table_hbm.at[idx_v.at[j]], buf, sem)

    def gather_wait(j, buf, sem):
        pltpu.make_async_copy(table_hbm.at[idx_v.at[j]], buf, sem).wait()

    def write_start(j, buf, sem):
        pltpu.async_copy(buf, out_hbm.at[pl.ds(base + j * CHUNK, CHUNK)], sem)

    def write_wait(j, buf, sem):
        pltpu.make_async_copy(
            buf, out_hbm.at[pl.ds(base + j * CHUNK, CHUNK)], sem).wait()

    def add_pe(j, buf):
        p0 = lax.rem(j * CHUNK, S)

        def row_step(k, p):
            for c in range(DV):
                sl = pl.ds(c * LANES, LANES)
                plsc.addupdate(buf.at[k, sl], pe_v[p, sl])
            p = p + 1
            return lax.select(p >= S, p - S, p)

        lax.fori_loop(0, CHUNK, row_step, p0, unroll=2)

    gather_start(0, buf0, gs0)

    def pair_step(i, carry):
        j0 = 2 * i
        j1 = j0 + 1
        gather_wait(j0, buf0, gs0)

        @pl.when(i > 0)
        def _():
            write_wait(j1 - 2, buf1, ws1)

        gather_start(j1, buf1, gs1)
        add_pe(j0, buf0)
        write_start(j0, buf0, ws0)
        gather_wait(j1, buf1, gs1)

        @pl.when(i < NPAIR - 1)
        def _():
            write_wait(j0, buf0, ws0)
            gather_start(j0 + 2, buf0, gs0)

        add_pe(j1, buf1)
        write_start(j1, buf1, ws1)
        return carry

    lax.fori_loop(0, NPAIR, pair_step, 0, unroll=False)
    write_wait(NCHUNKS - 2, buf0, ws0)
    write_wait(NCHUNKS - 1, buf1, ws1)


_sc_call = pl.kernel(
    _body,
    out_type=jax.ShapeDtypeStruct((ROWS, D), jnp.float32),
    mesh=_MESH,
    scratch_types=[
        pltpu.VMEM((NCHUNKS, CHUNK), jnp.int32),
        pltpu.VMEM((S, D), jnp.float32),
        pltpu.VMEM((CHUNK, D), jnp.float32),
        pltpu.VMEM((CHUNK, D), jnp.float32),
        pltpu.SemaphoreType.DMA,
        pltpu.SemaphoreType.DMA,
        pltpu.SemaphoreType.DMA,
        pltpu.SemaphoreType.DMA,
    ],
)


def kernel(input_ids, table):
    ids = input_ids.astype(jnp.int32).reshape(NW, NCHUNKS, CHUNK)
    pe = _make_pos_enc()
    out = _sc_call(ids, table, pe)
    return out.reshape(B, S, D)


# speedup vs baseline: 7.7365x; 2.2795x over previous
import math
import numpy as np
import jax
import jax.numpy as jnp
from jax import lax
from jax.experimental import pallas as pl
from jax.experimental.pallas import tpu as pltpu
from jax.experimental.pallas import tpu_sc as plsc

VOCAB = 100000
D = 128
S = 200
B = 4096

NC = 2
NS = 16
NW = NC * NS
ROWS = B * S
RPW = ROWS // NW
CHUNK = 128
NCHUNKS = RPW // CHUNK
LANES = 16
DV = D // LANES


def _make_pos_enc():
    pos_enc = np.zeros((S, D), dtype=np.float32)
    position = np.arange(0, S, dtype=np.float32)[:, None]
    div_term = np.exp(
        np.arange(0, D, 2, dtype=np.float32) * (-math.log(10000.0) / D))
    pos_enc[:, 0::2] = np.sin(position * div_term)
    pos_enc[:, 1::2] = np.cos(position * div_term)
    ext = np.concatenate([pos_enc, pos_enc[:CHUNK]], axis=0)
    return jnp.asarray(ext)


_MESH = plsc.VectorSubcoreMesh(core_axis_name="c", subcore_axis_name="s")


NPAIR = NCHUNKS // 2


def _body(ids_hbm, table_hbm, pe_hbm, out_hbm, idx_v, pe_sh, buf0, buf1,
          gs0, gs1, ws0, ws1, ps0, ps1):
    sid = lax.axis_index("s")
    wid = sid * NC + lax.axis_index("c")
    base = wid * RPW
    pltpu.sync_copy(ids_hbm.at[wid], idx_v)
    @pl.when(sid == 0)
    def _():
        pltpu.sync_copy(pe_hbm, pe_sh)

    plsc.subcore_barrier()

    def prefill_start(j, buf, sem):
        p0 = lax.rem(j * CHUNK, S)
        pltpu.async_copy(pe_sh.at[pl.ds(p0, CHUNK)], buf, sem)

    def prefill_wait(j, buf, sem):
        p0 = lax.rem(j * CHUNK, S)
        pltpu.make_async_copy(pe_sh.at[pl.ds(p0, CHUNK)], buf, sem).wait()

    def gather_start(j, buf, sem):
        pltpu.async_copy(table_hbm.at[idx_v.at[j]], buf, sem, add=True)

    def gather_wait(j, buf, sem):
        pltpu.make_async_copy(table_hbm.at[idx_v.at[j]], buf, sem).wait()

    def write_start(j, buf, sem):
        pltpu.async_copy(buf, out_hbm.at[pl.ds(base + j * CHUNK, CHUNK)], sem)

    def write_wait(j, buf, sem):
        pltpu.make_async_copy(
            buf, out_hbm.at[pl.ds(base + j * CHUNK, CHUNK)], sem).wait()

    prefill_start(0, buf0, ps0)
    prefill_wait(0, buf0, ps0)
    gather_start(0, buf0, gs0)

    def pair_step(i, carry):
        j0 = 2 * i
        j1 = j0 + 1
        @pl.when(i > 0)
        def _():
            write_wait(j1 - 2, buf1, ws1)

        prefill_start(j1, buf1, ps1)
        gather_wait(j0, buf0, gs0)
        prefill_wait(j1, buf1, ps1)
        gather_start(j1, buf1, gs1)
        write_start(j0, buf0, ws0)

        @pl.when(i < NPAIR - 1)
        def _():
            write_wait(j0, buf0, ws0)
            prefill_start(j0 + 2, buf0, ps0)

        gather_wait(j1, buf1, gs1)

        @pl.when(i < NPAIR - 1)
        def _():
            prefill_wait(j0 + 2, buf0, ps0)
            gather_start(j0 + 2, buf0, gs0)

        write_start(j1, buf1, ws1)
        return carry

    lax.fori_loop(0, NPAIR, pair_step, 0, unroll=False)
    write_wait(NCHUNKS - 2, buf0, ws0)
    write_wait(NCHUNKS - 1, buf1, ws1)


_sc_call = pl.kernel(
    _body,
    out_type=jax.ShapeDtypeStruct((ROWS, D), jnp.float32),
    mesh=_MESH,
    scratch_types=[
        pltpu.VMEM((NCHUNKS, CHUNK), jnp.int32),
        pltpu.VMEM_SHARED((S + CHUNK, D), jnp.float32),
        pltpu.VMEM((CHUNK, D), jnp.float32),
        pltpu.VMEM((CHUNK, D), jnp.float32),
        pltpu.SemaphoreType.DMA,
        pltpu.SemaphoreType.DMA,
        pltpu.SemaphoreType.DMA,
        pltpu.SemaphoreType.DMA,
        pltpu.SemaphoreType.DMA,
        pltpu.SemaphoreType.DMA,
    ],
)


def kernel(input_ids, table):
    ids = input_ids.astype(jnp.int32).reshape(NW, NCHUNKS, CHUNK)
    pe = _make_pos_enc()
    out = _sc_call(ids, table, pe)
    return out.reshape(B, S, D)


# speedup vs baseline: 9.1232x; 1.1792x over previous
import math
import numpy as np
import jax
import jax.numpy as jnp
from jax import lax
from jax.experimental import pallas as pl
from jax.experimental.pallas import tpu as pltpu
from jax.experimental.pallas import tpu_sc as plsc

VOCAB = 100000
D = 128
S = 200
B = 4096

NC = 2
NS = 16
NW = NC * NS
ROWS = B * S
RPW = ROWS // NW
CHUNK = 128
NCHUNKS = RPW // CHUNK
LANES = 16
DV = D // LANES


def _make_pos_enc():
    pos_enc = np.zeros((S, D), dtype=np.float32)
    position = np.arange(0, S, dtype=np.float32)[:, None]
    div_term = np.exp(
        np.arange(0, D, 2, dtype=np.float32) * (-math.log(10000.0) / D))
    pos_enc[:, 0::2] = np.sin(position * div_term)
    pos_enc[:, 1::2] = np.cos(position * div_term)
    ext = np.concatenate([pos_enc, pos_enc[:CHUNK]], axis=0)
    return jnp.asarray(ext)


_MESH = plsc.VectorSubcoreMesh(core_axis_name="c", subcore_axis_name="s")


NBUF = 5
NGROUP = NCHUNKS // NBUF


def _body(ids_hbm, table_hbm, pe_hbm, out_hbm, idx_v, pe_sh, bufs,
          psems, gsems, wsems):
    sid = lax.axis_index("s")
    wid = sid * NC + lax.axis_index("c")
    base = wid * RPW
    pltpu.sync_copy(ids_hbm.at[wid], idx_v)
    @pl.when(sid == 0)
    def _():
        pltpu.sync_copy(pe_hbm, pe_sh)

    plsc.subcore_barrier()

    def prefill_start(j, buf, sem):
        p0 = lax.rem(j * CHUNK, S)
        pltpu.async_copy(pe_sh.at[pl.ds(p0, CHUNK)], buf, sem)

    def prefill_wait(j, buf, sem):
        p0 = lax.rem(j * CHUNK, S)
        pltpu.make_async_copy(pe_sh.at[pl.ds(p0, CHUNK)], buf, sem).wait()

    def gather_start(j, buf, sem):
        pltpu.async_copy(table_hbm.at[idx_v.at[j]], buf, sem, add=True)

    def gather_wait(j, buf, sem):
        pltpu.make_async_copy(table_hbm.at[idx_v.at[j]], buf, sem).wait()

    def write_start(j, buf, sem):
        pltpu.async_copy(buf, out_hbm.at[pl.ds(base + j * CHUNK, CHUNK)], sem)

    def write_wait(j, buf, sem):
        pltpu.make_async_copy(
            buf, out_hbm.at[pl.ds(base + j * CHUNK, CHUNK)], sem).wait()

    def group_step(g, carry):
        for t in range(NBUF):
            j = g * NBUF + t

            @pl.when(g > 0)
            def _():
                write_wait(j - NBUF, bufs[t], wsems[t])

            prefill_start(j, bufs[t], psems[t])

            tp = (t - 1) % NBUF
            if t >= 1:
                prefill_wait(j - 1, bufs[tp], psems[tp])
                gather_start(j - 1, bufs[tp], gsems[tp])
            else:
                @pl.when(g > 0)
                def _():
                    prefill_wait(j - 1, bufs[tp], psems[tp])
                    gather_start(j - 1, bufs[tp], gsems[tp])

            tg = (t - 3) % NBUF
            if t >= 3:
                gather_wait(j - 3, bufs[tg], gsems[tg])
                write_start(j - 3, bufs[tg], wsems[tg])
            else:
                @pl.when(g > 0)
                def _():
                    gather_wait(j - 3, bufs[tg], gsems[tg])
                    write_start(j - 3, bufs[tg], wsems[tg])

        return carry

    lax.fori_loop(0, NGROUP, group_step, 0, unroll=False)

    last = NCHUNKS - 1
    prefill_wait(last, bufs[NBUF - 1], psems[NBUF - 1])
    gather_start(last, bufs[NBUF - 1], gsems[NBUF - 1])
    for j in (NCHUNKS - 3, NCHUNKS - 2, NCHUNKS - 1):
        b = j % NBUF
        gather_wait(j, bufs[b], gsems[b])
        write_start(j, bufs[b], wsems[b])
    for j in range(NCHUNKS - NBUF, NCHUNKS):
        b = j % NBUF
        write_wait(j, bufs[b], wsems[b])


_sc_call = pl.kernel(
    _body,
    out_type=jax.ShapeDtypeStruct((ROWS, D), jnp.float32),
    mesh=_MESH,
    scratch_types=[
        pltpu.VMEM((NCHUNKS, CHUNK), jnp.int32),
        pltpu.VMEM_SHARED((S + CHUNK, D), jnp.float32),
        tuple(pltpu.VMEM((CHUNK, D), jnp.float32) for _ in range(NBUF)),
        tuple(pltpu.SemaphoreType.DMA for _ in range(NBUF)),
        tuple(pltpu.SemaphoreType.DMA for _ in range(NBUF)),
        tuple(pltpu.SemaphoreType.DMA for _ in range(NBUF)),
    ],
)


def kernel(input_ids, table):
    ids = input_ids.astype(jnp.int32).reshape(NW, NCHUNKS, CHUNK)
    pe = _make_pos_enc()
    out = _sc_call(ids, table, pe)
    return out.reshape(B, S, D)


# speedup vs baseline: 9.1274x; 1.0005x over previous
import math
import numpy as np
import jax
import jax.numpy as jnp
from jax import lax
from jax.experimental import pallas as pl
from jax.experimental.pallas import tpu as pltpu
from jax.experimental.pallas import tpu_sc as plsc

VOCAB = 100000
D = 128
S = 200
B = 4096

NC = 2
NS = 16
NW = NC * NS
ROWS = B * S
RPW = ROWS // NW
CHUNK = 128
NCHUNKS = RPW // CHUNK
LANES = 16
DV = D // LANES


def _make_pos_enc():
    pos_enc = np.zeros((S, D), dtype=np.float32)
    position = np.arange(0, S, dtype=np.float32)[:, None]
    div_term = np.exp(
        np.arange(0, D, 2, dtype=np.float32) * (-math.log(10000.0) / D))
    pos_enc[:, 0::2] = np.sin(position * div_term)
    pos_enc[:, 1::2] = np.cos(position * div_term)
    ext = np.concatenate([pos_enc, pos_enc[:CHUNK]], axis=0)
    return jnp.asarray(ext)


_MESH = plsc.VectorSubcoreMesh(core_axis_name="c", subcore_axis_name="s")


NBUF = 5
NGROUP = NCHUNKS // NBUF


def _body(ids_hbm, table_hbm, pe_hbm, out_hbm, idx_v, pe_sh, bufs,
          psems, gsems, wsems):
    sid = lax.axis_index("s")
    wid = sid * NC + lax.axis_index("c")
    base = wid * RPW
    pltpu.sync_copy(ids_hbm.at[wid], idx_v)
    @pl.when(sid == 0)
    def _():
        pltpu.sync_copy(pe_hbm, pe_sh)

    plsc.subcore_barrier()

    def prefill_start(j, buf, sem):
        p0 = lax.rem(j * CHUNK, S)
        pltpu.async_copy(pe_sh.at[pl.ds(p0, CHUNK)], buf, sem)

    def prefill_wait(j, buf, sem):
        p0 = lax.rem(j * CHUNK, S)
        pltpu.make_async_copy(pe_sh.at[pl.ds(p0, CHUNK)], buf, sem).wait()

    def gather_start(j, buf, sem):
        pltpu.async_copy(table_hbm.at[idx_v.at[j]], buf, sem, add=True)

    def gather_wait(j, buf, sem):
        pltpu.make_async_copy(table_hbm.at[idx_v.at[j]], buf, sem).wait()

    def write_start(j, buf, sem):
        pltpu.async_copy(buf, out_hbm.at[pl.ds(base + j * CHUNK, CHUNK)], sem)

    def write_wait(j, buf, sem):
        pltpu.make_async_copy(
            buf, out_hbm.at[pl.ds(base + j * CHUNK, CHUNK)], sem).wait()

    def group_step(g, carry):
        for t in range(NBUF):
            j = g * NBUF + t

            @pl.when(g > 0)
            def _():
                write_wait(j - NBUF, bufs[t], wsems[t])

            prefill_start(j, bufs[t], psems[t])

            tp = (t - 1) % NBUF
            if t >= 1:
                prefill_wait(j - 1, bufs[tp], psems[tp])
                gather_start(j - 1, bufs[tp], gsems[tp])
            else:
                @pl.when(g > 0)
                def _():
                    prefill_wait(j - 1, bufs[tp], psems[tp])
                    gather_start(j - 1, bufs[tp], gsems[tp])

            tg = (t - 4) % NBUF
            if t >= 4:
                gather_wait(j - 4, bufs[tg], gsems[tg])
                write_start(j - 4, bufs[tg], wsems[tg])
            else:
                @pl.when(g > 0)
                def _():
                    gather_wait(j - 4, bufs[tg], gsems[tg])
                    write_start(j - 4, bufs[tg], wsems[tg])

        return carry

    lax.fori_loop(0, NGROUP, group_step, 0, unroll=False)

    last = NCHUNKS - 1
    prefill_wait(last, bufs[NBUF - 1], psems[NBUF - 1])
    gather_start(last, bufs[NBUF - 1], gsems[NBUF - 1])
    for j in (NCHUNKS - 4, NCHUNKS - 3, NCHUNKS - 2, NCHUNKS - 1):
        b = j % NBUF
        gather_wait(j, bufs[b], gsems[b])
        write_start(j, bufs[b], wsems[b])
    for j in range(NCHUNKS - NBUF, NCHUNKS):
        b = j % NBUF
        write_wait(j, bufs[b], wsems[b])


_sc_call = pl.kernel(
    _body,
    out_type=jax.ShapeDtypeStruct((ROWS, D), jnp.float32),
    mesh=_MESH,
    scratch_types=[
        pltpu.VMEM((NCHUNKS, CHUNK), jnp.int32),
        pltpu.VMEM_SHARED((S + CHUNK, D), jnp.float32),
        tuple(pltpu.VMEM((CHUNK, D), jnp.float32) for _ in range(NBUF)),
        tuple(pltpu.SemaphoreType.DMA for _ in range(NBUF)),
        tuple(pltpu.SemaphoreType.DMA for _ in range(NBUF)),
        tuple(pltpu.SemaphoreType.DMA for _ in range(NBUF)),
    ],
)


def kernel(input_ids, table):
    ids = input_ids.astype(jnp.int32).reshape(NW, NCHUNKS, CHUNK)
    pe = _make_pos_enc()
    out = _sc_call(ids, table, pe)
    return out.reshape(B, S, D)


# speedup vs baseline: 9.1510x; 1.0026x over previous
import math
import numpy as np
import jax
import jax.numpy as jnp
from jax import lax
from jax.experimental import pallas as pl
from jax.experimental.pallas import tpu as pltpu
from jax.experimental.pallas import tpu_sc as plsc

VOCAB = 100000
D = 128
S = 200
B = 4096

NC = 2
NS = 16
NW = NC * NS
ROWS = B * S
RPW = ROWS // NW
CHUNK = 128
NCHUNKS = RPW // CHUNK


def _make_pos_enc():
    pos_enc = np.zeros((S, D), dtype=np.float32)
    position = np.arange(0, S, dtype=np.float32)[:, None]
    div_term = np.exp(
        np.arange(0, D, 2, dtype=np.float32) * (-math.log(10000.0) / D))
    pos_enc[:, 0::2] = np.sin(position * div_term)
    pos_enc[:, 1::2] = np.cos(position * div_term)
    ext = np.concatenate([pos_enc, pos_enc[:CHUNK]], axis=0)
    return jnp.asarray(ext)


_MESH = plsc.VectorSubcoreMesh(core_axis_name="c", subcore_axis_name="s")


NBUF = 5
NGROUP = NCHUNKS // NBUF


def _body(ids_hbm, table_hbm, pe_hbm, out_hbm, idx_v, pe_sh, bufs,
          psems, gsems, wsems):
    sid = lax.axis_index("s")
    wid = sid * NC + lax.axis_index("c")
    base = wid * RPW
    pltpu.sync_copy(ids_hbm.at[wid], idx_v)
    @pl.when(sid == 0)
    def _():
        pltpu.sync_copy(pe_hbm, pe_sh)

    plsc.subcore_barrier()

    def prefill_start(j, buf, sem):
        p0 = lax.rem(j * CHUNK, S)
        pltpu.async_copy(pe_sh.at[pl.ds(p0, CHUNK)], buf, sem)

    def prefill_wait(j, buf, sem):
        p0 = lax.rem(j * CHUNK, S)
        pltpu.make_async_copy(pe_sh.at[pl.ds(p0, CHUNK)], buf, sem).wait()

    def gather_start(j, buf, sem):
        pltpu.async_copy(table_hbm.at[idx_v.at[j]], buf, sem, add=True)

    def gather_wait(j, buf, sem):
        pltpu.make_async_copy(table_hbm.at[idx_v.at[j]], buf, sem).wait()

    def write_start(j, buf, sem):
        pltpu.async_copy(buf, out_hbm.at[pl.ds(base + j * CHUNK, CHUNK)], sem)

    def write_wait(j, buf, sem):
        pltpu.make_async_copy(
            buf, out_hbm.at[pl.ds(base + j * CHUNK, CHUNK)], sem).wait()

    def group_step(g, carry):
        for t in range(NBUF):
            j = g * NBUF + t

            @pl.when(g > 0)
            def _():
                write_wait(j - NBUF, bufs[t], wsems[t])

            prefill_start(j, bufs[t], psems[t])

            tp = (t - 1) % NBUF
            if t >= 1:
                prefill_wait(j - 1, bufs[tp], psems[tp])
                gather_start(j - 1, bufs[tp], gsems[tp])
            else:
                @pl.when(g > 0)
                def _():
                    prefill_wait(j - 1, bufs[tp], psems[tp])
                    gather_start(j - 1, bufs[tp], gsems[tp])

            tg = (t - 4) % NBUF
            if t >= 4:
                gather_wait(j - 4, bufs[tg], gsems[tg])
                write_start(j - 4, bufs[tg], wsems[tg])
            else:
                @pl.when(g > 0)
                def _():
                    gather_wait(j - 4, bufs[tg], gsems[tg])
                    write_start(j - 4, bufs[tg], wsems[tg])

        return carry

    lax.fori_loop(0, NGROUP, group_step, 0, unroll=False)

    last = NCHUNKS - 1
    prefill_wait(last, bufs[NBUF - 1], psems[NBUF - 1])
    gather_start(last, bufs[NBUF - 1], gsems[NBUF - 1])
    for j in (NCHUNKS - 4, NCHUNKS - 3, NCHUNKS - 2, NCHUNKS - 1):
        b = j % NBUF
        gather_wait(j, bufs[b], gsems[b])
        write_start(j, bufs[b], wsems[b])
    for j in range(NCHUNKS - NBUF, NCHUNKS):
        b = j % NBUF
        write_wait(j, bufs[b], wsems[b])


_sc_call = pl.kernel(
    _body,
    out_type=jax.ShapeDtypeStruct((ROWS, D), jnp.float32),
    mesh=_MESH,
    scratch_types=[
        pltpu.VMEM((NCHUNKS, CHUNK), jnp.int32),
        pltpu.VMEM_SHARED((S + CHUNK, D), jnp.float32),
        tuple(pltpu.VMEM((CHUNK, D), jnp.float32) for _ in range(NBUF)),
        tuple(pltpu.SemaphoreType.DMA for _ in range(NBUF)),
        tuple(pltpu.SemaphoreType.DMA for _ in range(NBUF)),
        tuple(pltpu.SemaphoreType.DMA for _ in range(NBUF)),
    ],
)


def kernel(input_ids, table):
    ids = input_ids.astype(jnp.int32).reshape(NW, NCHUNKS, CHUNK)
    pe = _make_pos_enc()
    out = _sc_call(ids, table, pe)
    return out.reshape(B, S, D)
